# baseline (speedup 1.0000x reference)
import jax
import jax.numpy as jnp
from jax import lax
from jax.experimental import pallas as pl
from jax.experimental.pallas import tpu as pltpu

B = 1024
F = 4096
RB = 128
CB = 512
K_DROP = 103
N_SEL = 8


def _prep_body(m_ref, sq_ref, hi_ref, lo_ref):
    x = m_ref[...]
    sq_ref[...] = jnp.sum(x * x, axis=1, keepdims=True)
    hi = x.astype(jnp.bfloat16)
    hi_ref[...] = hi
    lo_ref[...] = (x - hi.astype(jnp.float32)).astype(jnp.bfloat16)


def _score_body(ahi_ref, alo_ref, bhi_ref, blo_ref, sqc_ref, sqr_ref,
                out_ref, dT_ref):
    j = pl.program_id(1)
    ahi = ahi_ref[...]
    alo = alo_ref[...]
    bhi = bhi_ref[...]
    blo = blo_ref[...]
    dn = (((1,), (1,)), ((), ()))
    g = (lax.dot_general(bhi, ahi, dn, preferred_element_type=jnp.float32)
         + lax.dot_general(bhi, alo, dn, preferred_element_type=jnp.float32)
         + lax.dot_general(blo, ahi, dn, preferred_element_type=jnp.float32))
    d2 = sqc_ref[...] + sqr_ref[0] - 2.0 * g
    d2 = jnp.maximum(d2, 0.0)
    safe = jnp.where(d2 > 0.0, d2, 1.0)
    d = jnp.where(d2 > 0.0, jnp.sqrt(safe), 0.0)
    dT_ref[pl.ds(j * RB, RB), :] = d

    @pl.when(j == (B // RB) - 1)
    def _select():
        dall = dT_ref[...]
        bits = lax.bitcast_convert_type(dall, jnp.int32)
        rowsum = jnp.sum(dall, axis=0, keepdims=True)
        rowmin = jnp.min(dall, axis=0, keepdims=True)
        ones_row = jnp.ones((1, B), jnp.bfloat16)

        def bit_step(t, T):
            bit = jnp.int32(30) - t
            cand = T | (jnp.int32(1) << bit)
            ind = jnp.where(bits >= cand, 1.0, 0.0).astype(jnp.bfloat16)
            cnt = lax.dot_general(ones_row, ind, (((1,), (0,)), ((), ())),
                                  preferred_element_type=jnp.float32)
            return jnp.where(cnt >= K_DROP, cand, T)

        T = lax.fori_loop(0, 31, bit_step, jnp.zeros((1, RB), jnp.int32))
        gt = bits > T
        cnt_gt = jnp.sum(jnp.where(gt, 1.0, 0.0), axis=0, keepdims=True)
        sum_gt = jnp.sum(jnp.where(gt, dall, 0.0), axis=0, keepdims=True)
        kth = lax.bitcast_convert_type(T, jnp.float32)
        sumtop = sum_gt + (K_DROP - cnt_gt) * kth
        out_ref[0] = rowsum - sumtop - rowmin


def _select_body(scores_ref, m_ref, out_ref, w_ref):
    c = pl.program_id(0)

    @pl.when(c == 0)
    def _weights():
        s = scores_ref[...]
        iota = lax.broadcasted_iota(jnp.int32, (B, 1), 0)
        w = jnp.zeros((B, 1), jnp.float32)

        def pick(_, carry):
            s, w = carry
            m = jnp.min(s)
            elig = s == m
            idx = jnp.min(jnp.where(elig, iota, jnp.int32(2 ** 30)))
            onehot = iota == idx
            w = w + jnp.where(onehot, 1.0 / N_SEL, 0.0)
            s = jnp.where(onehot, jnp.float32(jnp.inf), s)
            return s, w

        _, w = lax.fori_loop(0, N_SEL, pick, (s, w))
        w_ref[...] = w

    out_ref[0] = jnp.sum(m_ref[...] * w_ref[...], axis=0, keepdims=True)


def kernel(matrix):
    rowsq, mhi, mlo = pl.pallas_call(
        _prep_body,
        grid=(B // RB,),
        in_specs=[pl.BlockSpec((RB, F), lambda i: (i, 0))],
        out_specs=[
            pl.BlockSpec((RB, 1), lambda i: (i, 0)),
            pl.BlockSpec((RB, F), lambda i: (i, 0)),
            pl.BlockSpec((RB, F), lambda i: (i, 0)),
        ],
        out_shape=[
            jax.ShapeDtypeStruct((B, 1), jnp.float32),
            jax.ShapeDtypeStruct((B, F), jnp.bfloat16),
            jax.ShapeDtypeStruct((B, F), jnp.bfloat16),
        ],
    )(matrix)

    sq_row3 = rowsq.reshape(B // RB, 1, RB)

    scores3 = pl.pallas_call(
        _score_body,
        grid=(B // RB, B // RB),
        in_specs=[
            pl.BlockSpec((RB, F), lambda i, j: (i, 0)),
            pl.BlockSpec((RB, F), lambda i, j: (i, 0)),
            pl.BlockSpec((RB, F), lambda i, j: (j, 0)),
            pl.BlockSpec((RB, F), lambda i, j: (j, 0)),
            pl.BlockSpec((RB, 1), lambda i, j: (j, 0)),
            pl.BlockSpec((1, 1, RB), lambda i, j: (i, 0, 0)),
        ],
        out_specs=pl.BlockSpec((1, 1, RB), lambda i, j: (i, 0, 0)),
        out_shape=jax.ShapeDtypeStruct((B // RB, 1, RB), jnp.float32),
        scratch_shapes=[pltpu.VMEM((B, RB), jnp.float32)],
    )(mhi, mlo, mhi, mlo, rowsq, sq_row3)

    scores = scores3.reshape(B, 1)

    out3 = pl.pallas_call(
        _select_body,
        grid=(F // CB,),
        in_specs=[
            pl.BlockSpec((B, 1), lambda c: (0, 0)),
            pl.BlockSpec((B, CB), lambda c: (0, c)),
        ],
        out_specs=pl.BlockSpec((1, 1, CB), lambda c: (0, 0, c)),
        out_shape=jax.ShapeDtypeStruct((1, 1, F // CB * CB), jnp.float32),
        scratch_shapes=[pltpu.VMEM((B, 1), jnp.float32)],
    )(scores, matrix)

    return out3.reshape(F)


# speedup vs baseline: 1.0253x; 1.0253x over previous
import jax
import jax.numpy as jnp
from jax import lax
from jax.experimental import pallas as pl
from jax.experimental.pallas import tpu as pltpu

B = 1024
F = 4096
RB = 128
CB = 512
K_DROP = 103
N_SEL = 8


def _prep_body(m_ref, sq_ref, hi_ref, lo_ref):
    x = m_ref[...]
    sq_ref[...] = jnp.sum(x * x, axis=1, keepdims=True)
    hi = x.astype(jnp.bfloat16)
    hi_ref[...] = hi
    lo_ref[...] = (x - hi.astype(jnp.float32)).astype(jnp.bfloat16)


def _score_body(ahi_ref, alo_ref, bhi_ref, blo_ref, sqc_ref, sqr_ref,
                out_ref, dT_ref):
    j = pl.program_id(1)
    ahi = ahi_ref[...]
    alo = alo_ref[...]
    bhi = bhi_ref[...]
    blo = blo_ref[...]
    dn = (((1,), (1,)), ((), ()))
    g = (lax.dot_general(bhi, ahi, dn, preferred_element_type=jnp.float32)
         + lax.dot_general(bhi, alo, dn, preferred_element_type=jnp.float32)
         + lax.dot_general(blo, ahi, dn, preferred_element_type=jnp.float32))
    d2 = sqc_ref[...] + sqr_ref[0] - 2.0 * g
    d2 = jnp.maximum(d2, 0.0)
    safe = jnp.where(d2 > 0.0, d2, 1.0)
    d = jnp.where(d2 > 0.0, jnp.sqrt(safe), 0.0)
    dT_ref[pl.ds(j * RB, RB), :] = d

    @pl.when(j == (B // RB) - 1)
    def _select():
        dall = dT_ref[...]
        bits = lax.bitcast_convert_type(dall, jnp.int32)
        rowsum = jnp.sum(dall, axis=0, keepdims=True)
        rowmin = jnp.min(dall, axis=0, keepdims=True)
        ones_row = jnp.ones((1, B), jnp.bfloat16)

        def bit_step(t, T):
            bit = jnp.int32(30) - t
            cand = T | (jnp.int32(1) << bit)
            ge = bits >= cand
            cnt = jnp.sum(jnp.where(ge, 1, 0), axis=0, keepdims=True)
            return jnp.where(cnt >= K_DROP, cand, T)

        T = lax.fori_loop(0, 31, bit_step, jnp.zeros((1, RB), jnp.int32))
        gt = bits > T
        cnt_gt = jnp.sum(jnp.where(gt, 1.0, 0.0), axis=0, keepdims=True)
        sum_gt = jnp.sum(jnp.where(gt, dall, 0.0), axis=0, keepdims=True)
        kth = lax.bitcast_convert_type(T, jnp.float32)
        sumtop = sum_gt + (K_DROP - cnt_gt) * kth
        out_ref[0] = rowsum - sumtop - rowmin


def _select_body(scores_ref, m_ref, out_ref, w_ref):
    c = pl.program_id(0)

    @pl.when(c == 0)
    def _weights():
        s = scores_ref[...]
        iota = lax.broadcasted_iota(jnp.int32, (B, 1), 0)
        w = jnp.zeros((B, 1), jnp.float32)

        def pick(_, carry):
            s, w = carry
            m = jnp.min(s)
            elig = s == m
            idx = jnp.min(jnp.where(elig, iota, jnp.int32(2 ** 30)))
            onehot = iota == idx
            w = w + jnp.where(onehot, 1.0 / N_SEL, 0.0)
            s = jnp.where(onehot, jnp.float32(jnp.inf), s)
            return s, w

        _, w = lax.fori_loop(0, N_SEL, pick, (s, w))
        w_ref[...] = w

    out_ref[0] = jnp.sum(m_ref[...] * w_ref[...], axis=0, keepdims=True)


def kernel(matrix):
    rowsq, mhi, mlo = pl.pallas_call(
        _prep_body,
        grid=(B // RB,),
        in_specs=[pl.BlockSpec((RB, F), lambda i: (i, 0))],
        out_specs=[
            pl.BlockSpec((RB, 1), lambda i: (i, 0)),
            pl.BlockSpec((RB, F), lambda i: (i, 0)),
            pl.BlockSpec((RB, F), lambda i: (i, 0)),
        ],
        out_shape=[
            jax.ShapeDtypeStruct((B, 1), jnp.float32),
            jax.ShapeDtypeStruct((B, F), jnp.bfloat16),
            jax.ShapeDtypeStruct((B, F), jnp.bfloat16),
        ],
    )(matrix)

    sq_row3 = rowsq.reshape(B // RB, 1, RB)

    scores3 = pl.pallas_call(
        _score_body,
        grid=(B // RB, B // RB),
        in_specs=[
            pl.BlockSpec((RB, F), lambda i, j: (i, 0)),
            pl.BlockSpec((RB, F), lambda i, j: (i, 0)),
            pl.BlockSpec((RB, F), lambda i, j: (j, 0)),
            pl.BlockSpec((RB, F), lambda i, j: (j, 0)),
            pl.BlockSpec((RB, 1), lambda i, j: (j, 0)),
            pl.BlockSpec((1, 1, RB), lambda i, j: (i, 0, 0)),
        ],
        out_specs=pl.BlockSpec((1, 1, RB), lambda i, j: (i, 0, 0)),
        out_shape=jax.ShapeDtypeStruct((B // RB, 1, RB), jnp.float32),
        scratch_shapes=[pltpu.VMEM((B, RB), jnp.float32)],
    )(mhi, mlo, mhi, mlo, rowsq, sq_row3)

    scores = scores3.reshape(B, 1)

    out3 = pl.pallas_call(
        _select_body,
        grid=(F // CB,),
        in_specs=[
            pl.BlockSpec((B, 1), lambda c: (0, 0)),
            pl.BlockSpec((B, CB), lambda c: (0, c)),
        ],
        out_specs=pl.BlockSpec((1, 1, CB), lambda c: (0, 0, c)),
        out_shape=jax.ShapeDtypeStruct((1, 1, F // CB * CB), jnp.float32),
        scratch_shapes=[pltpu.VMEM((B, 1), jnp.float32)],
    )(scores, matrix)

    return out3.reshape(F)


# speedup vs baseline: 1.7905x; 1.7464x over previous
import jax
import jax.numpy as jnp
from jax import lax
from jax.experimental import pallas as pl
from jax.experimental.pallas import tpu as pltpu

B = 1024
F = 4096
RB = 128
CB = 512
K_DROP = 103
N_SEL = 8


def _prep_body(m_ref, sq_ref):
    x = m_ref[...]
    sq_ref[...] = jnp.sum(x * x, axis=1, keepdims=True)


def _score_body(m_ref, a_ref, sqc_ref, sqr_ref, out_ref):
    a = a_ref[...]
    m = m_ref[...]
    g = lax.dot_general(m, a, (((1,), (1,)), ((), ())),
                        preferred_element_type=jnp.float32)
    d2 = sqc_ref[...] + sqr_ref[0] - 2.0 * g
    d2 = jnp.maximum(d2, 0.0)
    safe = jnp.where(d2 > 0.0, d2, 1.0)
    dall = jnp.where(d2 > 0.0, jnp.sqrt(safe), 0.0)

    bits = lax.bitcast_convert_type(dall, jnp.int32)
    rowsum = jnp.sum(dall, axis=0, keepdims=True)
    rowmin = jnp.min(dall, axis=0, keepdims=True)

    def bit_step(t, T):
        bit = jnp.int32(30) - t
        cand = T | (jnp.int32(1) << bit)
        ge = bits >= cand
        cnt = jnp.sum(jnp.where(ge, 1, 0), axis=0, keepdims=True)
        return jnp.where(cnt >= K_DROP, cand, T)

    T = lax.fori_loop(0, 31, bit_step, jnp.zeros((1, RB), jnp.int32))
    gt = bits > T
    cnt_gt = jnp.sum(jnp.where(gt, 1.0, 0.0), axis=0, keepdims=True)
    sum_gt = jnp.sum(jnp.where(gt, dall, 0.0), axis=0, keepdims=True)
    kth = lax.bitcast_convert_type(T, jnp.float32)
    sumtop = sum_gt + (K_DROP - cnt_gt) * kth
    out_ref[0] = rowsum - sumtop - rowmin


def _select_body(scores_ref, m_ref, out_ref, w_ref):
    c = pl.program_id(0)

    @pl.when(c == 0)
    def _weights():
        s = scores_ref[...]
        iota = lax.broadcasted_iota(jnp.int32, (B, 1), 0)
        w = jnp.zeros((B, 1), jnp.float32)

        def pick(_, carry):
            s, w = carry
            m = jnp.min(s)
            elig = s == m
            idx = jnp.min(jnp.where(elig, iota, jnp.int32(2 ** 30)))
            onehot = iota == idx
            w = w + jnp.where(onehot, 1.0 / N_SEL, 0.0)
            s = jnp.where(onehot, jnp.float32(jnp.inf), s)
            return s, w

        _, w = lax.fori_loop(0, N_SEL, pick, (s, w))
        w_ref[...] = w

    out_ref[0] = jnp.sum(m_ref[...] * w_ref[...], axis=0, keepdims=True)


def kernel(matrix):
    rowsq = pl.pallas_call(
        _prep_body,
        grid=(B // RB,),
        in_specs=[pl.BlockSpec((RB, F), lambda i: (i, 0))],
        out_specs=pl.BlockSpec((RB, 1), lambda i: (i, 0)),
        out_shape=jax.ShapeDtypeStruct((B, 1), jnp.float32),
    )(matrix)

    sq_row3 = rowsq.reshape(B // RB, 1, RB)

    scores3 = pl.pallas_call(
        _score_body,
        grid=(B // RB,),
        in_specs=[
            pl.BlockSpec((B, F), lambda i: (0, 0)),
            pl.BlockSpec((RB, F), lambda i: (i, 0)),
            pl.BlockSpec((B, 1), lambda i: (0, 0)),
            pl.BlockSpec((1, 1, RB), lambda i: (i, 0, 0)),
        ],
        out_specs=pl.BlockSpec((1, 1, RB), lambda i: (i, 0, 0)),
        out_shape=jax.ShapeDtypeStruct((B // RB, 1, RB), jnp.float32),
    )(matrix, matrix, rowsq, sq_row3)

    scores = scores3.reshape(B, 1)

    out3 = pl.pallas_call(
        _select_body,
        grid=(F // CB,),
        in_specs=[
            pl.BlockSpec((B, 1), lambda c: (0, 0)),
            pl.BlockSpec((B, CB), lambda c: (0, c)),
        ],
        out_specs=pl.BlockSpec((1, 1, CB), lambda c: (0, 0, c)),
        out_shape=jax.ShapeDtypeStruct((1, 1, F // CB * CB), jnp.float32),
        scratch_shapes=[pltpu.VMEM((B, 1), jnp.float32)],
    )(scores, matrix)

    return out3.reshape(F)


# speedup vs baseline: 1.9590x; 1.0941x over previous
import jax
import jax.numpy as jnp
from jax import lax
from jax.experimental import pallas as pl
from jax.experimental.pallas import tpu as pltpu

B = 1024
F = 4096
RB = 128
NB = B // RB
K_DROP = 103
N_SEL = 8


def _prep_body(m_ref, sq_ref):
    x = m_ref[...]
    sq_ref[...] = jnp.sum(x * x, axis=1, keepdims=True)


def _krum_body(m_ref, sqc_ref, sqr_i_ref, out_ref, dcol_ref, sc_ref):
    i = pl.program_id(0)
    j = pl.program_id(1)

    @pl.when(j >= i)
    def _matmul():
        mi = m_ref[pl.ds(i * RB, RB), :]
        mj = m_ref[pl.ds(j * RB, RB), :]
        g = lax.dot_general(mj, mi, (((1,), (1,)), ((), ())),
                            preferred_element_type=jnp.float32)
        sq_j = sqc_ref[pl.ds(j * RB, RB), :]
        d2 = sq_j + sqr_i_ref[0] - 2.0 * g
        d2 = jnp.maximum(d2, 0.0)
        safe = jnp.where(d2 > 0.0, d2, 1.0)
        d = jnp.where(d2 > 0.0, jnp.sqrt(safe), 0.0)
        dcol_ref[i, pl.ds(j * RB, RB), :] = d

        @pl.when(j > i)
        def _mirror():
            dcol_ref[j, pl.ds(i * RB, RB), :] = d.T

    @pl.when(j == NB - 1)
    def _score():
        dall = dcol_ref[i]
        bits = lax.bitcast_convert_type(dall, jnp.int32)
        rowsum = jnp.sum(dall, axis=0, keepdims=True)
        rowmin = jnp.min(dall, axis=0, keepdims=True)

        def bit_step(t, T):
            bit = jnp.int32(30) - t
            cand = T | (jnp.int32(1) << bit)
            ge = bits >= cand
            cnt = jnp.sum(jnp.where(ge, 1, 0), axis=0, keepdims=True)
            return jnp.where(cnt >= K_DROP, cand, T)

        T = lax.fori_loop(0, 31, bit_step, jnp.zeros((1, RB), jnp.int32))
        gt = bits > T
        cnt_gt = jnp.sum(jnp.where(gt, 1.0, 0.0), axis=0, keepdims=True)
        sum_gt = jnp.sum(jnp.where(gt, dall, 0.0), axis=0, keepdims=True)
        kth = lax.bitcast_convert_type(T, jnp.float32)
        sumtop = sum_gt + (K_DROP - cnt_gt) * kth
        sc_ref[i] = rowsum - sumtop - rowmin

    @pl.when((i == NB - 1) & (j == NB - 1))
    def _select():
        s = sc_ref[...].reshape(NB, RB)
        iota = (lax.broadcasted_iota(jnp.int32, (NB, RB), 0) * RB
                + lax.broadcasted_iota(jnp.int32, (NB, RB), 1))
        w0 = jnp.zeros((NB, RB), jnp.float32)

        def pick(_, carry):
            s, w = carry
            m = jnp.min(s)
            elig = s == m
            idx = jnp.min(jnp.where(elig, iota, jnp.int32(2 ** 30)))
            onehot = iota == idx
            w = w + jnp.where(onehot, 1.0 / N_SEL, 0.0)
            s = jnp.where(onehot, jnp.float32(jnp.inf), s)
            return s, w

        _, w = lax.fori_loop(0, N_SEL, pick, (s, w0))
        acc = jnp.zeros((1, F), jnp.float32)
        for ib in range(NB):
            acc = acc + lax.dot_general(
                w[ib:ib + 1, :], m_ref[ib * RB:(ib + 1) * RB, :],
                (((1,), (0,)), ((), ())),
                preferred_element_type=jnp.float32)
        out_ref[0] = acc


def kernel(matrix):
    rowsq = pl.pallas_call(
        _prep_body,
        grid=(NB,),
        in_specs=[pl.BlockSpec((RB, F), lambda i: (i, 0))],
        out_specs=pl.BlockSpec((RB, 1), lambda i: (i, 0)),
        out_shape=jax.ShapeDtypeStruct((B, 1), jnp.float32),
    )(matrix)

    sq_row3 = rowsq.reshape(NB, 1, RB)

    out3 = pl.pallas_call(
        _krum_body,
        grid=(NB, NB),
        in_specs=[
            pl.BlockSpec((B, F), lambda i, j: (0, 0)),
            pl.BlockSpec((B, 1), lambda i, j: (0, 0)),
            pl.BlockSpec((1, 1, RB), lambda i, j: (i, 0, 0)),
        ],
        out_specs=pl.BlockSpec((1, 1, F), lambda i, j: (0, 0, 0)),
        out_shape=jax.ShapeDtypeStruct((1, 1, F), jnp.float32),
        scratch_shapes=[
            pltpu.VMEM((NB, B, RB), jnp.float32),
            pltpu.VMEM((NB, 1, RB), jnp.float32),
        ],
    )(matrix, rowsq, sq_row3)

    return out3.reshape(F)


# speedup vs baseline: 3.2654x; 1.6669x over previous
import jax
import jax.numpy as jnp
from jax import lax
from jax.experimental import pallas as pl
from jax.experimental.pallas import tpu as pltpu

B = 1024
F = 4096
RB = 256
NB = B // RB
K_DROP = 103
N_SEL = 8


def _prep_body(m_ref, sq_ref):
    x = m_ref[...]
    sq_ref[...] = jnp.sum(x * x, axis=1, keepdims=True)


def _krum_body(m_ref, sqc_ref, sqr_i_ref, out_ref, dcol_ref, sc_ref):
    i = pl.program_id(0)
    j = pl.program_id(1)

    @pl.when(j >= i)
    def _matmul():
        mi = m_ref[pl.ds(i * RB, RB), :]
        mj = m_ref[pl.ds(j * RB, RB), :]
        g = lax.dot_general(mj, mi, (((1,), (1,)), ((), ())),
                            preferred_element_type=jnp.float32)
        sq_j = sqc_ref[pl.ds(j * RB, RB), :]
        d2 = sq_j + sqr_i_ref[0] - 2.0 * g
        d2 = jnp.maximum(d2, 0.0)
        safe = jnp.where(d2 > 0.0, d2, 1.0)
        d = jnp.where(d2 > 0.0, jnp.sqrt(safe), 0.0)
        dcol_ref[i, pl.ds(j * RB, RB), :] = d

        @pl.when(j > i)
        def _mirror():
            dcol_ref[j, pl.ds(i * RB, RB), :] = d.T

    @pl.when(j == NB - 1)
    def _score():
        dall = dcol_ref[i]
        bits = lax.bitcast_convert_type(dall, jnp.int32)
        rowsum = jnp.sum(dall, axis=0, keepdims=True)
        rowmin = jnp.min(dall, axis=0, keepdims=True)

        def bit_step(t, T):
            bit = jnp.int32(30) - t
            cand = T | (jnp.int32(1) << bit)
            ge = bits >= cand
            cnt = jnp.sum(jnp.where(ge, 1, 0), axis=0, keepdims=True)
            return jnp.where(cnt >= K_DROP, cand, T)

        T = lax.fori_loop(0, 31, bit_step, jnp.zeros((1, RB), jnp.int32))
        gt = bits > T
        cnt_gt = jnp.sum(jnp.where(gt, 1.0, 0.0), axis=0, keepdims=True)
        sum_gt = jnp.sum(jnp.where(gt, dall, 0.0), axis=0, keepdims=True)
        kth = lax.bitcast_convert_type(T, jnp.float32)
        sumtop = sum_gt + (K_DROP - cnt_gt) * kth
        sc_ref[i] = rowsum - sumtop - rowmin

    @pl.when((i == NB - 1) & (j == NB - 1))
    def _select():
        s = sc_ref[...].reshape(NB, RB)
        iota = (lax.broadcasted_iota(jnp.int32, (NB, RB), 0) * RB
                + lax.broadcasted_iota(jnp.int32, (NB, RB), 1))
        w0 = jnp.zeros((NB, RB), jnp.float32)

        def pick(_, carry):
            s, w = carry
            m = jnp.min(s)
            elig = s == m
            idx = jnp.min(jnp.where(elig, iota, jnp.int32(2 ** 30)))
            onehot = iota == idx
            w = w + jnp.where(onehot, 1.0 / N_SEL, 0.0)
            s = jnp.where(onehot, jnp.float32(jnp.inf), s)
            return s, w

        _, w = lax.fori_loop(0, N_SEL, pick, (s, w0))
        acc = jnp.zeros((1, F), jnp.float32)
        for ib in range(NB):
            acc = acc + lax.dot_general(
                w[ib:ib + 1, :], m_ref[ib * RB:(ib + 1) * RB, :],
                (((1,), (0,)), ((), ())),
                preferred_element_type=jnp.float32)
        out_ref[0] = acc


def kernel(matrix):
    rowsq = pl.pallas_call(
        _prep_body,
        grid=(NB,),
        in_specs=[pl.BlockSpec((RB, F), lambda i: (i, 0))],
        out_specs=pl.BlockSpec((RB, 1), lambda i: (i, 0)),
        out_shape=jax.ShapeDtypeStruct((B, 1), jnp.float32),
    )(matrix)

    sq_row3 = rowsq.reshape(NB, 1, RB)

    out3 = pl.pallas_call(
        _krum_body,
        grid=(NB, NB),
        in_specs=[
            pl.BlockSpec((B, F), lambda i, j: (0, 0)),
            pl.BlockSpec((B, 1), lambda i, j: (0, 0)),
            pl.BlockSpec((1, 1, RB), lambda i, j: (i, 0, 0)),
        ],
        out_specs=pl.BlockSpec((1, 1, F), lambda i, j: (0, 0, 0)),
        out_shape=jax.ShapeDtypeStruct((1, 1, F), jnp.float32),
        scratch_shapes=[
            pltpu.VMEM((NB, B, RB), jnp.float32),
            pltpu.VMEM((NB, 1, RB), jnp.float32),
        ],
    )(matrix, rowsq, sq_row3)

    return out3.reshape(F)


# speedup vs baseline: 3.6226x; 1.1094x over previous
import jax
import jax.numpy as jnp
from jax import lax
from jax.experimental import pallas as pl
from jax.experimental.pallas import tpu as pltpu

B = 1024
F = 4096
RB = 1024
NB = B // RB
K_DROP = 103
N_SEL = 8


def _prep_body(m_ref, sq_ref):
    x = m_ref[...]
    sq_ref[...] = jnp.sum(x * x, axis=1, keepdims=True)


def _krum_body(m_ref, sqc_ref, sqr_i_ref, out_ref, dcol_ref, sc_ref):
    i = pl.program_id(0)
    j = pl.program_id(1)

    @pl.when(j >= i)
    def _matmul():
        mi = m_ref[pl.ds(i * RB, RB), :]
        mj = m_ref[pl.ds(j * RB, RB), :]
        g = lax.dot_general(mj, mi, (((1,), (1,)), ((), ())),
                            preferred_element_type=jnp.float32)
        sq_j = sqc_ref[pl.ds(j * RB, RB), :]
        d2 = sq_j + sqr_i_ref[0] - 2.0 * g
        d2 = jnp.maximum(d2, 0.0)
        safe = jnp.where(d2 > 0.0, d2, 1.0)
        d = jnp.where(d2 > 0.0, jnp.sqrt(safe), 0.0)
        dcol_ref[i, pl.ds(j * RB, RB), :] = d

        @pl.when(j > i)
        def _mirror():
            dcol_ref[j, pl.ds(i * RB, RB), :] = d.T

    @pl.when(j == NB - 1)
    def _score():
        dall = dcol_ref[i]
        bits = lax.bitcast_convert_type(dall, jnp.int32)
        rowsum = jnp.sum(dall, axis=0, keepdims=True)
        rowmin = jnp.min(dall, axis=0, keepdims=True)

        def bit_step(t, T):
            bit = jnp.int32(30) - t
            cand = T | (jnp.int32(1) << bit)
            ge = bits >= cand
            cnt = jnp.sum(jnp.where(ge, 1, 0), axis=0, keepdims=True)
            return jnp.where(cnt >= K_DROP, cand, T)

        T = lax.fori_loop(0, 31, bit_step, jnp.zeros((1, RB), jnp.int32))
        gt = bits > T
        cnt_gt = jnp.sum(jnp.where(gt, 1.0, 0.0), axis=0, keepdims=True)
        sum_gt = jnp.sum(jnp.where(gt, dall, 0.0), axis=0, keepdims=True)
        kth = lax.bitcast_convert_type(T, jnp.float32)
        sumtop = sum_gt + (K_DROP - cnt_gt) * kth
        sc_ref[i] = rowsum - sumtop - rowmin

    @pl.when((i == NB - 1) & (j == NB - 1))
    def _select():
        s = sc_ref[...].reshape(NB, RB)
        iota = (lax.broadcasted_iota(jnp.int32, (NB, RB), 0) * RB
                + lax.broadcasted_iota(jnp.int32, (NB, RB), 1))
        w0 = jnp.zeros((NB, RB), jnp.float32)

        def pick(_, carry):
            s, w = carry
            m = jnp.min(s)
            elig = s == m
            idx = jnp.min(jnp.where(elig, iota, jnp.int32(2 ** 30)))
            onehot = iota == idx
            w = w + jnp.where(onehot, 1.0 / N_SEL, 0.0)
            s = jnp.where(onehot, jnp.float32(jnp.inf), s)
            return s, w

        _, w = lax.fori_loop(0, N_SEL, pick, (s, w0))
        acc = jnp.zeros((1, F), jnp.float32)
        for ib in range(NB):
            acc = acc + lax.dot_general(
                w[ib:ib + 1, :], m_ref[ib * RB:(ib + 1) * RB, :],
                (((1,), (0,)), ((), ())),
                preferred_element_type=jnp.float32)
        out_ref[0] = acc


def kernel(matrix):
    rowsq = pl.pallas_call(
        _prep_body,
        grid=(NB,),
        in_specs=[pl.BlockSpec((RB, F), lambda i: (i, 0))],
        out_specs=pl.BlockSpec((RB, 1), lambda i: (i, 0)),
        out_shape=jax.ShapeDtypeStruct((B, 1), jnp.float32),
    )(matrix)

    sq_row3 = rowsq.reshape(NB, 1, RB)

    out3 = pl.pallas_call(
        _krum_body,
        grid=(NB, NB),
        in_specs=[
            pl.BlockSpec((B, F), lambda i, j: (0, 0)),
            pl.BlockSpec((B, 1), lambda i, j: (0, 0)),
            pl.BlockSpec((1, 1, RB), lambda i, j: (i, 0, 0)),
        ],
        out_specs=pl.BlockSpec((1, 1, F), lambda i, j: (0, 0, 0)),
        out_shape=jax.ShapeDtypeStruct((1, 1, F), jnp.float32),
        scratch_shapes=[
            pltpu.VMEM((NB, B, RB), jnp.float32),
            pltpu.VMEM((NB, 1, RB), jnp.float32),
        ],
    )(matrix, rowsq, sq_row3)

    return out3.reshape(F)


# speedup vs baseline: 4.5105x; 1.2451x over previous
import jax
import jax.numpy as jnp
from jax import lax
from jax.experimental import pallas as pl
from jax.experimental.pallas import tpu as pltpu

B = 1024
F = 4096
RB = 512
NB = B // RB
K_DROP = 103
N_SEL = 8


def _krum_body(m_ref, out_ref, dcol_ref, sc_ref, sqc_ref, sqr_ref):
    i = pl.program_id(0)
    j = pl.program_id(1)

    @pl.when((i == 0) & (j == 0))
    def _prep():
        m = m_ref[...]
        msq = m * m
        sqc_ref[...] = jnp.sum(msq, axis=1, keepdims=True)
        sqr_ref[...] = lax.dot_general(
            jnp.ones((1, F), jnp.float32), msq, (((1,), (1,)), ((), ())),
            preferred_element_type=jnp.float32)

    @pl.when(j >= i)
    def _matmul():
        mi = m_ref[pl.ds(i * RB, RB), :]
        mj = m_ref[pl.ds(j * RB, RB), :]
        g = lax.dot_general(mj, mi, (((1,), (1,)), ((), ())),
                            preferred_element_type=jnp.float32)
        sq_j = sqc_ref[pl.ds(j * RB, RB), :]
        sq_i = sqr_ref[:, pl.ds(i * RB, RB)]
        d2 = sq_j + sq_i - 2.0 * g
        d2 = jnp.maximum(d2, 0.0)
        safe = jnp.where(d2 > 0.0, d2, 1.0)
        d = jnp.where(d2 > 0.0, jnp.sqrt(safe), 0.0)
        dcol_ref[i, pl.ds(j * RB, RB), :] = d

        @pl.when(j > i)
        def _mirror():
            dcol_ref[j, pl.ds(i * RB, RB), :] = d.T

    @pl.when(j == NB - 1)
    def _score():
        dall = dcol_ref[i]
        bits = lax.bitcast_convert_type(dall, jnp.int32)
        rowsum = jnp.sum(dall, axis=0, keepdims=True)
        rowmin = jnp.min(dall, axis=0, keepdims=True)

        def bit_step(t, T):
            bit = jnp.int32(30) - t
            cand = T | (jnp.int32(1) << bit)
            ge = bits >= cand
            cnt = jnp.sum(jnp.where(ge, 1, 0), axis=0, keepdims=True)
            return jnp.where(cnt >= K_DROP, cand, T)

        T = lax.fori_loop(0, 31, bit_step, jnp.zeros((1, RB), jnp.int32))
        gt = bits > T
        cnt_gt = jnp.sum(jnp.where(gt, 1.0, 0.0), axis=0, keepdims=True)
        sum_gt = jnp.sum(jnp.where(gt, dall, 0.0), axis=0, keepdims=True)
        kth = lax.bitcast_convert_type(T, jnp.float32)
        sumtop = sum_gt + (K_DROP - cnt_gt) * kth
        sc_ref[i] = rowsum - sumtop - rowmin

    @pl.when((i == NB - 1) & (j == NB - 1))
    def _select():
        s = sc_ref[...].reshape(NB, RB)
        iota = (lax.broadcasted_iota(jnp.int32, (NB, RB), 0) * RB
                + lax.broadcasted_iota(jnp.int32, (NB, RB), 1))
        w0 = jnp.zeros((NB, RB), jnp.float32)

        def pick(_, carry):
            s, w = carry
            m = jnp.min(s)
            elig = s == m
            idx = jnp.min(jnp.where(elig, iota, jnp.int32(2 ** 30)))
            onehot = iota == idx
            w = w + jnp.where(onehot, 1.0 / N_SEL, 0.0)
            s = jnp.where(onehot, jnp.float32(jnp.inf), s)
            return s, w

        _, w = lax.fori_loop(0, N_SEL, pick, (s, w0))
        acc = jnp.zeros((1, F), jnp.float32)
        for ib in range(NB):
            acc = acc + lax.dot_general(
                w[ib:ib + 1, :], m_ref[ib * RB:(ib + 1) * RB, :],
                (((1,), (0,)), ((), ())),
                preferred_element_type=jnp.float32)
        out_ref[0] = acc


def kernel(matrix):
    out3 = pl.pallas_call(
        _krum_body,
        grid=(NB, NB),
        in_specs=[
            pl.BlockSpec((B, F), lambda i, j: (0, 0)),
        ],
        out_specs=pl.BlockSpec((1, 1, F), lambda i, j: (0, 0, 0)),
        out_shape=jax.ShapeDtypeStruct((1, 1, F), jnp.float32),
        scratch_shapes=[
            pltpu.VMEM((NB, B, RB), jnp.float32),
            pltpu.VMEM((NB, 1, RB), jnp.float32),
            pltpu.VMEM((B, 1), jnp.float32),
            pltpu.VMEM((1, B), jnp.float32),
        ],
    )(matrix)

    return out3.reshape(F)


# speedup vs baseline: 4.5397x; 1.0065x over previous
import jax
import jax.numpy as jnp
from jax import lax
from jax.experimental import pallas as pl
from jax.experimental.pallas import tpu as pltpu

B = 1024
F = 4096
RB = 512
NB = B // RB
K_DROP = 103
N_SEL = 8


def _sum_i16(x):
    r = x.shape[0]
    while r > 1:
        h = r // 2
        x = x[:h] + x[h:r]
        r = h
    return x


def _krum_body(m_ref, out_ref, dcol_ref, sc_ref, sqc_ref, sqr_ref):
    i = pl.program_id(0)
    j = pl.program_id(1)

    @pl.when((i == 0) & (j == 0))
    def _prep():
        m = m_ref[...]
        msq = m * m
        sqc_ref[...] = jnp.sum(msq, axis=1, keepdims=True)
        sqr_ref[...] = lax.dot_general(
            jnp.ones((1, F), jnp.float32), msq, (((1,), (1,)), ((), ())),
            preferred_element_type=jnp.float32)

    @pl.when(j >= i)
    def _matmul():
        mi = m_ref[pl.ds(i * RB, RB), :]
        mj = m_ref[pl.ds(j * RB, RB), :]
        g = lax.dot_general(mj, mi, (((1,), (1,)), ((), ())),
                            preferred_element_type=jnp.float32)
        sq_j = sqc_ref[pl.ds(j * RB, RB), :]
        sq_i = sqr_ref[:, pl.ds(i * RB, RB)]
        d2 = sq_j + sq_i - 2.0 * g
        d2 = jnp.maximum(d2, 0.0)
        safe = jnp.where(d2 > 0.0, d2, 1.0)
        d = jnp.where(d2 > 0.0, jnp.sqrt(safe), 0.0)
        dcol_ref[i, pl.ds(j * RB, RB), :] = d

        @pl.when(j > i)
        def _mirror():
            dcol_ref[j, pl.ds(i * RB, RB), :] = d.T

    @pl.when(j == NB - 1)
    def _score():
        dall = dcol_ref[i]
        bits = lax.bitcast_convert_type(dall, jnp.int32)
        rowsum = jnp.sum(dall, axis=0, keepdims=True)
        rowmin = jnp.min(dall, axis=0, keepdims=True)

        h16 = lax.shift_right_logical(bits, 16).astype(jnp.int16)
        lx16 = (bits ^ jnp.int32(0x8000)).astype(jnp.int16)
        one16 = jnp.int16(1)
        zero16 = jnp.int16(0)
        kdrop16 = jnp.int16(K_DROP)

        P = jnp.zeros((1, RB), jnp.int16)
        for b in range(14, -1, -1):
            cand = P | jnp.int16(1 << b)
            cnt = _sum_i16(jnp.where(h16 >= cand, one16, zero16))
            P = jnp.where(cnt >= kdrop16, cand, P)

        maskP = h16 == P
        c_hi = _sum_i16(jnp.where(h16 > P, one16, zero16))
        k_rem = kdrop16 - c_hi

        Lx = jnp.full((1, RB), jnp.int16(-0x8000))
        for b in range(15, -1, -1):
            if b == 15:
                cand = Lx & jnp.int16(0x7FFF)
            else:
                cand = Lx | jnp.int16(1 << b)
            hit = maskP & (lx16 >= cand)
            cnt = _sum_i16(jnp.where(hit, one16, zero16))
            Lx = jnp.where(cnt >= k_rem, cand, Lx)

        low_u = (Lx.astype(jnp.int32) ^ jnp.int32(0x8000)) & jnp.int32(0xFFFF)
        T = lax.shift_left(P.astype(jnp.int32), 16) | low_u
        gt = bits > T
        cnt_gt = jnp.sum(jnp.where(gt, 1.0, 0.0), axis=0, keepdims=True)
        sum_gt = jnp.sum(jnp.where(gt, dall, 0.0), axis=0, keepdims=True)
        kth = lax.bitcast_convert_type(T, jnp.float32)
        sumtop = sum_gt + (K_DROP - cnt_gt) * kth
        sc_ref[i] = rowsum - sumtop - rowmin

    @pl.when((i == NB - 1) & (j == NB - 1))
    def _select():
        s = sc_ref[...].reshape(NB, RB)
        iota = (lax.broadcasted_iota(jnp.int32, (NB, RB), 0) * RB
                + lax.broadcasted_iota(jnp.int32, (NB, RB), 1))
        w0 = jnp.zeros((NB, RB), jnp.float32)

        def pick(_, carry):
            s, w = carry
            m = jnp.min(s)
            elig = s == m
            idx = jnp.min(jnp.where(elig, iota, jnp.int32(2 ** 30)))
            onehot = iota == idx
            w = w + jnp.where(onehot, 1.0 / N_SEL, 0.0)
            s = jnp.where(onehot, jnp.float32(jnp.inf), s)
            return s, w

        _, w = lax.fori_loop(0, N_SEL, pick, (s, w0))
        acc = jnp.zeros((1, F), jnp.float32)
        for ib in range(NB):
            acc = acc + lax.dot_general(
                w[ib:ib + 1, :], m_ref[ib * RB:(ib + 1) * RB, :],
                (((1,), (0,)), ((), ())),
                preferred_element_type=jnp.float32)
        out_ref[0] = acc


def kernel(matrix):
    out3 = pl.pallas_call(
        _krum_body,
        grid=(NB, NB),
        in_specs=[
            pl.BlockSpec((B, F), lambda i, j: (0, 0)),
        ],
        out_specs=pl.BlockSpec((1, 1, F), lambda i, j: (0, 0, 0)),
        out_shape=jax.ShapeDtypeStruct((1, 1, F), jnp.float32),
        scratch_shapes=[
            pltpu.VMEM((NB, B, RB), jnp.float32),
            pltpu.VMEM((NB, 1, RB), jnp.float32),
            pltpu.VMEM((B, 1), jnp.float32),
            pltpu.VMEM((1, B), jnp.float32),
        ],
    )(matrix)

    return out3.reshape(F)
